# baseline (speedup 1.0000x reference)
import functools

import jax
import jax.numpy as jnp
from jax.experimental import pallas as pl
from jax.experimental.pallas import tpu as pltpu

_BLK = 4096
_INV_SQRT2 = 0.7071067811865476


def _mlp_block(props_ref, w1_ref, b1_ref, w2_ref, b2_ref, te_ref, out_ref):
    p = props_ref[:, 0:1]
    pm = jnp.concatenate([p, jnp.ones_like(p)], axis=1)
    wcat = jnp.concatenate([w1_ref[...], b1_ref[...]], axis=0)
    hs = jnp.dot(pm, _INV_SQRT2 * wcat,
                 preferred_element_type=jnp.float32)
    g = hs + hs * jax.lax.erf(hs)
    out = jnp.dot(g, _INV_SQRT2 * w2_ref[...],
                  preferred_element_type=jnp.float32)
    out = out + (b2_ref[0, :] + te_ref[0, :])[None, :]
    valid = jnp.logical_not(jnp.isnan(p))
    out_ref[...] = jnp.where(valid, out, 0.0)


@functools.partial(jax.jit, static_argnames=())
def kernel(idx, props, W1, b1, W2, b2, type_emb, type_index):
    del idx, type_index
    b = props.shape[0]
    two_n = W1.shape[1]
    n = W2.shape[1]

    grid = (b // _BLK,)
    out = pl.pallas_call(
        _mlp_block,
        grid=grid,
        in_specs=[
            pl.BlockSpec((_BLK, 1), lambda i: (i, 0)),
            pl.BlockSpec((1, two_n), lambda i: (0, 0)),
            pl.BlockSpec((1, two_n), lambda i: (0, 0)),
            pl.BlockSpec((two_n, n), lambda i: (0, 0)),
            pl.BlockSpec((1, n), lambda i: (0, 0)),
            pl.BlockSpec((1, n), lambda i: (0, 0)),
        ],
        out_specs=pl.BlockSpec((_BLK, n), lambda i: (i, 0)),
        out_shape=jax.ShapeDtypeStruct((b, n), jnp.float32),
        compiler_params=pltpu.CompilerParams(
            dimension_semantics=("parallel",)),
    )(props, W1, b1.reshape(1, two_n), W2, b2.reshape(1, n), type_emb)
    return out.reshape(b, 1, n)
